# baseline (speedup 1.0000x reference)
import functools

import jax
import jax.numpy as jnp
from jax import lax
from jax.experimental import pallas as pl
from jax.experimental.pallas import tpu as pltpu
from jax.experimental.pallas import tpu_sc as plsc

VOCAB = 1000000
EMB = 32
HID = 128
OUT = 16
BATCH = 16384

GRP = 4
NSLAB = VOCAB // GRP
NC = 2
NS = 16
NW = NC * NS
B_PER_W = BATCH // NW
CHUNK = 128
NCH = B_PER_W // CHUNK

CH_OUT = 200
CH_IN = CH_OUT * GRP
NCHK = NSLAB // CH_OUT
CHK_FULL = NCHK // NW
CHK_REM = NCHK - CHK_FULL * NW


def _sc_repack_body(emb_hbm, out_hbm, buf, buf2, sem):
    c = lax.axis_index("c")
    s = lax.axis_index("s")
    wid = s * NC + c

    def do_chunk(ci):
        in0 = pl.multiple_of(ci * CH_IN, 8)
        out0 = pl.multiple_of(ci * CH_OUT, 8)
        pltpu.sync_copy(emb_hbm.at[pl.ds(in0, CH_IN), :], buf)

        @plsc.parallel_loop(0, CH_OUT, 1, unroll=2)
        def _(q):
            for k in range(GRP):
                for h in range(EMB // 16):
                    buf2[q, pl.ds(k * EMB + h * 16, 16)] = buf[
                        q * GRP + k, pl.ds(h * 16, 16)
                    ]

        pltpu.sync_copy(buf2, out_hbm.at[pl.ds(out0, CH_OUT), :])

    def body(i, carry):
        do_chunk(wid + i * NW)
        return carry

    lax.fori_loop(0, CHK_FULL, body, 0)

    @pl.when(wid < CHK_REM)
    def _():
        do_chunk(CHK_FULL * NW + wid)


@jax.jit
def _sc_repack(emb):
    mesh = plsc.VectorSubcoreMesh(core_axis_name="c", subcore_axis_name="s")
    k = functools.partial(
        pl.kernel,
        mesh=mesh,
        out_type=jax.ShapeDtypeStruct((NSLAB, GRP * EMB), jnp.float32),
        scratch_types=[
            pltpu.VMEM((CH_IN, EMB), jnp.float32),
            pltpu.VMEM((CH_OUT, GRP * EMB), jnp.float32),
            pltpu.SemaphoreType.DMA,
        ],
    )(_sc_repack_body)
    return k(emb)


def _sc_gather_body(emb2_hbm, idx_hbm, out_hbm, idx_v, slabs_v, sem):
    c = lax.axis_index("c")
    s = lax.axis_index("s")
    wid = s * NC + c
    pltpu.sync_copy(idx_hbm.at[wid], idx_v)
    cps = [
        pltpu.async_copy(emb2_hbm.at[idx_v.at[j]], slabs_v.at[j], sem)
        for j in range(NCH)
    ]
    for cp in cps:
        cp.wait()
    pltpu.sync_copy(slabs_v, out_hbm.at[wid])


@jax.jit
def _sc_gather(emb2, idx):
    mesh = plsc.VectorSubcoreMesh(core_axis_name="c", subcore_axis_name="s")
    k = functools.partial(
        pl.kernel,
        mesh=mesh,
        out_type=jax.ShapeDtypeStruct((NW, NCH, CHUNK, GRP * EMB), jnp.float32),
        scratch_types=[
            pltpu.VMEM((NCH, CHUNK), jnp.int32),
            pltpu.VMEM((NCH, CHUNK, GRP * EMB), jnp.float32),
            pltpu.SemaphoreType.DMA,
        ],
    )(_sc_gather_body)
    return k(emb2, idx)


def _mlp_body(h_ref, s_ref, w1_ref, b1_ref, w2_ref, b2_ref, w3_ref, b3_ref, o_ref):
    hw = h_ref[...]
    sel = s_ref[...]
    h = jnp.zeros((hw.shape[0], EMB), jnp.float32)
    for k in range(GRP):
        h = h + jnp.where(sel == k, hw[:, k * EMB : (k + 1) * EMB], 0.0)
    z = jnp.dot(h, w1_ref[...], preferred_element_type=jnp.float32)
    z = jnp.maximum(z + b1_ref[...], 0.0)
    z = jnp.dot(z, w2_ref[...], preferred_element_type=jnp.float32)
    z = jnp.maximum(z + b2_ref[...], 0.0)
    z = jnp.dot(z, w3_ref[...], preferred_element_type=jnp.float32)
    o_ref[...] = z + b3_ref[...]


BLK = 2048


@jax.jit
def _tc_mlp(hw, sel, W1, b1, W2, b2, W3, b3):
    grid = (BATCH // BLK,)
    full = lambda shape: pl.BlockSpec(shape, lambda i: (0, 0))
    return pl.pallas_call(
        _mlp_body,
        grid=grid,
        in_specs=[
            pl.BlockSpec((BLK, GRP * EMB), lambda i: (i, 0)),
            pl.BlockSpec((BLK, 1), lambda i: (i, 0)),
            full((EMB, HID)),
            full((1, HID)),
            full((HID, HID)),
            full((1, HID)),
            full((HID, OUT)),
            full((1, OUT)),
        ],
        out_specs=pl.BlockSpec((BLK, OUT), lambda i: (i, 0)),
        out_shape=jax.ShapeDtypeStruct((BATCH, OUT), jnp.float32),
    )(hw, sel, W1, b1, W2, b2, W3, b3)


def kernel(x, emb, W1, b1, W2, b2, W3, b3):
    xi = x.astype(jnp.int32)
    emb2 = _sc_repack(emb)
    gidx = (xi // GRP).reshape(NW, NCH, CHUNK)
    slabs = _sc_gather(emb2, gidx).reshape(BATCH, GRP * EMB)
    sel = (xi % GRP).reshape(BATCH, 1)
    return _tc_mlp(
        slabs,
        sel,
        W1,
        b1.reshape(1, HID),
        W2,
        b2.reshape(1, HID),
        W3,
        b3.reshape(1, OUT),
    )


# speedup vs baseline: 1.5028x; 1.5028x over previous
import functools

import jax
import jax.numpy as jnp
from jax import lax
from jax.experimental import pallas as pl
from jax.experimental.pallas import tpu as pltpu
from jax.experimental.pallas import tpu_sc as plsc

VOCAB = 1000000
EMB = 32
HID = 128
OUT = 16
BATCH = 16384

NTC = 8192
NSC = BATCH - NTC
NC = 2
NS = 16
NW = NC * NS
B_PER_W = NSC // NW
K = 16
LAG = 8


def _sc_gather_body(emb_hbm, idx_hbm, out_hbm, idx_v, idx_s, rows_v, sem):
    c = lax.axis_index("c")
    s = lax.axis_index("s")
    wid = s * NC + c
    base = wid * B_PER_W
    pltpu.sync_copy(idx_hbm.at[pl.ds(base, B_PER_W)], idx_v)

    @plsc.parallel_loop(0, B_PER_W // K, 1, unroll=1)
    def _(i):
        off = i * K
        vec = idx_v[pl.ds(off, K)]
        for k in range(K):
            r = jnp.sum(jnp.where(lax.iota(jnp.int32, K) == k, vec, 0))
            idx_s[off + k] = r

    def fire(off):
        for k in range(K):
            pltpu.async_copy(
                emb_hbm.at[pl.ds(idx_s[off + k], 1), :],
                rows_v.at[pl.ds(off + k, 1), :],
                sem,
            )

    def drain(off):
        pltpu.make_async_copy(
            emb_hbm.at[pl.ds(0, K)], rows_v.at[pl.ds(off, K)], sem
        ).wait()

    ngrp = B_PER_W // K
    for g in range(LAG):
        fire(g * K)

    def grp(i, carry):
        fire((i + LAG) * K)
        drain(i * K)
        return carry

    lax.fori_loop(0, ngrp - LAG, grp, 0)
    for g in range(ngrp - LAG, ngrp):
        drain(g * K)
    pltpu.sync_copy(rows_v, out_hbm.at[pl.ds(base, B_PER_W)])


@jax.jit
def _sc_gather(emb, idx):
    mesh = plsc.VectorSubcoreMesh(core_axis_name="c", subcore_axis_name="s")
    k = functools.partial(
        pl.kernel,
        mesh=mesh,
        out_type=jax.ShapeDtypeStruct((NSC, EMB), jnp.float32),
        scratch_types=[
            pltpu.VMEM((B_PER_W,), jnp.int32),
            pltpu.SMEM((B_PER_W,), jnp.int32),
            pltpu.VMEM((B_PER_W, EMB), jnp.float32),
            pltpu.SemaphoreType.DMA,
        ],
        compiler_params=pltpu.CompilerParams(needs_layout_passes=False),
    )(_sc_gather_body)
    return k(emb, idx)


GBLK = 128


def _tc_gather_body(idx_s, emb_ref, o_ref, sem):
    i = pl.program_id(0)
    cps = []
    for k in range(GBLK):
        r = idx_s[i * GBLK + k]
        cps.append(
            pltpu.make_async_copy(
                emb_ref.at[pl.ds(r, 1), :], o_ref.at[pl.ds(k, 1), :], sem
            )
        )
    for cp in cps:
        cp.start()
    for cp in cps:
        cp.wait()


@jax.jit
def _tc_gather(idx, emb):
    grid_spec = pltpu.PrefetchScalarGridSpec(
        num_scalar_prefetch=1,
        grid=(NTC // GBLK,),
        in_specs=[pl.BlockSpec(memory_space=pltpu.MemorySpace.HBM)],
        out_specs=pl.BlockSpec((GBLK, EMB), lambda i, idx_ref: (i, 0)),
        scratch_shapes=[pltpu.SemaphoreType.DMA],
    )
    return pl.pallas_call(
        _tc_gather_body,
        grid_spec=grid_spec,
        out_shape=jax.ShapeDtypeStruct((NTC, EMB), jnp.float32),
    )(idx, emb)


def _mlp_body(h_ref, w1_ref, b1_ref, w2_ref, b2_ref, w3_ref, b3_ref, o_ref):
    h = h_ref[...]
    z = jnp.dot(h, w1_ref[...], preferred_element_type=jnp.float32)
    z = jnp.maximum(z + b1_ref[...], 0.0)
    z = jnp.dot(z, w2_ref[...], preferred_element_type=jnp.float32)
    z = jnp.maximum(z + b2_ref[...], 0.0)
    z = jnp.dot(z, w3_ref[...], preferred_element_type=jnp.float32)
    o_ref[...] = z + b3_ref[...]


BLK = 2048


@jax.jit
def _tc_mlp(h, W1, b1, W2, b2, W3, b3):
    grid = (BATCH // BLK,)
    full = lambda shape: pl.BlockSpec(shape, lambda i: (0, 0))
    return pl.pallas_call(
        _mlp_body,
        grid=grid,
        in_specs=[
            pl.BlockSpec((BLK, EMB), lambda i: (i, 0)),
            full((EMB, HID)),
            full((1, HID)),
            full((HID, HID)),
            full((1, HID)),
            full((HID, OUT)),
            full((1, OUT)),
        ],
        out_specs=pl.BlockSpec((BLK, OUT), lambda i: (i, 0)),
        out_shape=jax.ShapeDtypeStruct((BATCH, OUT), jnp.float32),
    )(h, W1, b1, W2, b2, W3, b3)


def kernel(x, emb, W1, b1, W2, b2, W3, b3):
    xi = x.astype(jnp.int32)
    rows_sc = _sc_gather(emb, xi[:NSC])
    rows_tc = _tc_gather(xi[NSC:], emb)
    rows = jnp.concatenate([rows_sc, rows_tc], axis=0)
    return _tc_mlp(
        rows,
        W1,
        b1.reshape(1, HID),
        W2,
        b2.reshape(1, HID),
        W3,
        b3.reshape(1, OUT),
    )


# speedup vs baseline: 1.8799x; 1.2509x over previous
import functools

import jax
import jax.numpy as jnp
from jax import lax
from jax.experimental import pallas as pl
from jax.experimental.pallas import tpu as pltpu
from jax.experimental.pallas import tpu_sc as plsc

VOCAB = 1000000
EMB = 32
HID = 128
OUT = 16
BATCH = 16384

GRP = 8
SLAB = GRP * EMB
NSLAB = VOCAB // GRP
NC = 2
NS = 16
NW = NC * NS
B_PER_W = BATCH // NW
CHUNK = 128
NCH = B_PER_W // CHUNK
HALF = NCH // 2


K = 16


LAG = 8


def _sc_gather_body(emb_hbm, idx_hbm, out_hbm, idx_v, idx_s, rows_v, sem):
    c = lax.axis_index("c")
    s = lax.axis_index("s")
    wid = s * NC + c
    base = wid * B_PER_W
    pltpu.sync_copy(idx_hbm.at[pl.ds(base, B_PER_W)], idx_v)

    @plsc.parallel_loop(0, B_PER_W // K, 1, unroll=1)
    def _(i):
        off = i * K
        vec = idx_v[pl.ds(off, K)]
        for k in range(K):
            r = jnp.sum(jnp.where(lax.iota(jnp.int32, K) == k, vec, 0))
            idx_s[off + k] = r

    def fire(off):
        for k in range(K):
            pltpu.async_copy(
                emb_hbm.at[pl.ds(idx_s[off + k], 1), :],
                rows_v.at[pl.ds(off + k, 1), :],
                sem,
            )

    def drain(off):
        pltpu.make_async_copy(
            emb_hbm.at[pl.ds(0, K)], rows_v.at[pl.ds(off, K)], sem
        ).wait()

    ngrp = B_PER_W // K
    for g in range(LAG):
        fire(g * K)

    def grp(i, carry):
        fire((i + LAG) * K)
        drain(i * K)
        return carry

    lax.fori_loop(0, ngrp - LAG, grp, 0)
    for g in range(ngrp - LAG, ngrp):
        drain(g * K)
    pltpu.sync_copy(rows_v, out_hbm.at[pl.ds(base, B_PER_W)])


@jax.jit
def _sc_gather(emb, idx):
    mesh = plsc.VectorSubcoreMesh(core_axis_name="c", subcore_axis_name="s")
    k = functools.partial(
        pl.kernel,
        mesh=mesh,
        out_type=jax.ShapeDtypeStruct((BATCH, EMB), jnp.float32),
        scratch_types=[
            pltpu.VMEM((B_PER_W,), jnp.int32),
            pltpu.SMEM((B_PER_W,), jnp.int32),
            pltpu.VMEM((B_PER_W, EMB), jnp.float32),
            pltpu.SemaphoreType.DMA,
        ],
        compiler_params=pltpu.CompilerParams(needs_layout_passes=False),
    )(_sc_gather_body)
    return k(emb, idx)


RROWS = 1000


def _mlp_body(h_ref, w1_ref, b1_ref, w2_ref, b2_ref, w3_ref, b3_ref, o_ref):
    h = h_ref[...]
    z = jnp.dot(h, w1_ref[...], preferred_element_type=jnp.float32)
    z = jnp.maximum(z + b1_ref[...], 0.0)
    z = jnp.dot(z, w2_ref[...], preferred_element_type=jnp.float32)
    z = jnp.maximum(z + b2_ref[...], 0.0)
    z = jnp.dot(z, w3_ref[...], preferred_element_type=jnp.float32)
    o_ref[...] = z + b3_ref[...]


BLK = 2048


@jax.jit
def _tc_mlp(h, W1, b1, W2, b2, W3, b3):
    grid = (BATCH // BLK,)
    full = lambda shape: pl.BlockSpec(shape, lambda i: (0, 0))
    return pl.pallas_call(
        _mlp_body,
        grid=grid,
        in_specs=[
            pl.BlockSpec((BLK, EMB), lambda i: (i, 0)),
            full((EMB, HID)),
            full((1, HID)),
            full((HID, HID)),
            full((1, HID)),
            full((HID, OUT)),
            full((1, OUT)),
        ],
        out_specs=pl.BlockSpec((BLK, OUT), lambda i: (i, 0)),
        out_shape=jax.ShapeDtypeStruct((BATCH, OUT), jnp.float32),
    )(h, W1, b1, W2, b2, W3, b3)


def kernel(x, emb, W1, b1, W2, b2, W3, b3):
    rows = _sc_gather(emb, x.astype(jnp.int32))
    return _tc_mlp(
        rows,
        W1,
        b1.reshape(1, HID),
        W2,
        b2.reshape(1, HID),
        W3,
        b3.reshape(1, OUT),
    )
